# baseline (speedup 1.0000x reference)
import jax
import jax.numpy as jnp
from jax.experimental import pallas as pl
from jax.experimental.pallas import tpu as pltpu

_B, _C, _H, _W = 8, 384, 32, 32
_N = _H * _W
_K = 9


def _min_fold(a, target_rows):
    r = a.shape[0]
    while r > target_rows:
        h = r // 2
        a = jnp.minimum(a[:h], a[h:r])
        r = h
    return a


def _fused_kernel(x_ref, w_ref, b_ref, gamma_ref, beta_ref,
                  out_ref, y_scr, stats_scr):
    step = pl.program_id(0)

    @pl.when(step < _B)
    def _knn_linear():
        X = x_ref[0].reshape(_C, _N)
        W = w_ref[...]
        XX = X * X
        sqrow = jnp.sum(XX, axis=0, keepdims=True)
        sqcol = jnp.transpose(sqrow)
        G = jax.lax.dot_general(
            X, X, (((0,), (0,)), ((), ())),
            precision=jax.lax.Precision.DEFAULT,
            preferred_element_type=jnp.float32)
        S = sqcol - 2.0 * G
        for _ in range(_K):
            mn = jnp.min(_min_fold(S, 8), axis=0, keepdims=True)
            S = jnp.where(S == mn, jnp.inf, S)
        acc = jnp.isinf(S).astype(jnp.float32)
        WX = jax.lax.dot_general(
            W, X, (((1,), (0,)), ((), ())),
            precision=jax.lax.Precision.DEFAULT,
            preferred_element_type=jnp.float32)
        y = jax.lax.dot_general(
            WX, acc, (((1,), (0,)), ((), ())),
            precision=jax.lax.Precision.DEFAULT,
            preferred_element_type=jnp.float32) * (1.0 / _K) \
            + jnp.transpose(b_ref[...])
        y_scr[step] = y
        stats_scr[step, :, 0:1] = jnp.sum(y, axis=1, keepdims=True)
        stats_scr[step, :, 1:2] = jnp.sum(y * y, axis=1, keepdims=True)

    @pl.when(step >= _B)
    def _bn():
        s = jnp.sum(stats_scr[...], axis=0)
        cnt = float(_B * _N)
        mean = s[:, 0:1] * (1.0 / cnt)
        msq = s[:, 1:2] * (1.0 / cnt)
        var = msq - mean * mean
        scale = jax.lax.rsqrt(var + 1e-5) * jnp.transpose(gamma_ref[...])
        shift = jnp.transpose(beta_ref[...]) - mean * scale
        yj = y_scr[step - _B] * scale + shift
        out_ref[0] = jnp.maximum(yj.reshape(_C, _H, _W) + x_ref[0], 0.0)


def kernel(x, W_lin, b_lin, gamma, beta):
    out = pl.pallas_call(
        _fused_kernel,
        grid=(2 * _B,),
        in_specs=[
            pl.BlockSpec((1, _C, _H, _W),
                         lambda s: (jax.lax.rem(s, _B), 0, 0, 0)),
            pl.BlockSpec((_C, _C), lambda s: (0, 0)),
            pl.BlockSpec((1, _C), lambda s: (0, 0)),
            pl.BlockSpec((1, _C), lambda s: (0, 0)),
            pl.BlockSpec((1, _C), lambda s: (0, 0)),
        ],
        out_specs=pl.BlockSpec((1, _C, _H, _W),
                               lambda s: (jnp.maximum(s - _B, 0), 0, 0, 0)),
        out_shape=jax.ShapeDtypeStruct((_B, _C, _H, _W), jnp.float32),
        scratch_shapes=[
            pltpu.VMEM((_B, _C, _N), jnp.float32),
            pltpu.VMEM((_B, _C, 2), jnp.float32),
        ],
    )(x, W_lin, b_lin.reshape(1, _C), gamma.reshape(1, _C),
      beta.reshape(1, _C))
    return out


# speedup vs baseline: 2.0455x; 2.0455x over previous
import jax
import jax.numpy as jnp
from jax.experimental import pallas as pl
from jax.experimental.pallas import tpu as pltpu

_B, _C, _H, _W = 8, 384, 32, 32
_N = _H * _W
_K = 9


def _min_fold(a, target_rows):
    r = a.shape[0]
    while r > target_rows:
        h = r // 2
        a = jnp.minimum(a[:h], a[h:r])
        r = h
    return a


def _fused_kernel(x_ref, w_ref, b_ref, gamma_ref, beta_ref, xbn_ref,
                  out_ref, y_scr, stats_scr):
    step = pl.program_id(0)

    @pl.when(step < _B)
    def _knn_linear():
        X = x_ref[0]
        W = w_ref[...]
        XX = X * X
        sqrow = jnp.sum(XX, axis=0, keepdims=True)
        sqcol = jnp.transpose(sqrow)
        G = jax.lax.dot_general(
            X, X, (((0,), (0,)), ((), ())),
            precision=jax.lax.Precision.DEFAULT,
            preferred_element_type=jnp.float32)
        miota = jax.lax.broadcasted_iota(jnp.int32, (_N, _N), 0)
        niota = jax.lax.broadcasted_iota(jnp.int32, (_N, _N), 1)
        S = jnp.where(miota == niota, jnp.inf, sqcol - 2.0 * G)
        for _ in range(_K - 1):
            mn = jnp.min(_min_fold(S, 8), axis=0, keepdims=True)
            S = jnp.where(S == mn, jnp.inf, S)
        acc = jnp.isinf(S).astype(jnp.float32)
        WX = jax.lax.dot_general(
            W, X, (((1,), (0,)), ((), ())),
            precision=jax.lax.Precision.DEFAULT,
            preferred_element_type=jnp.float32)
        y = jax.lax.dot_general(
            WX, acc, (((1,), (0,)), ((), ())),
            precision=jax.lax.Precision.DEFAULT,
            preferred_element_type=jnp.float32) * (1.0 / _K) \
            + jnp.transpose(b_ref[...])
        y_scr[step] = y
        stats_scr[step, :, 0:1] = jnp.sum(y, axis=1, keepdims=True)
        stats_scr[step, :, 1:2] = jnp.sum(y * y, axis=1, keepdims=True)

    @pl.when(step >= _B)
    def _bn():
        s = jnp.sum(stats_scr[...], axis=0)
        cnt = float(_B * _N)
        mean = s[:, 0:1] * (1.0 / cnt)
        msq = s[:, 1:2] * (1.0 / cnt)
        var = msq - mean * mean
        scale = jax.lax.rsqrt(var + 1e-5) * jnp.transpose(gamma_ref[...])
        shift = jnp.transpose(beta_ref[...]) - mean * scale
        base = (step - _B) * 2
        for j in range(2):
            yj = y_scr[base + j]
            out_ref[j] = jnp.maximum(yj * scale + shift + xbn_ref[j], 0.0)


def kernel(x, W_lin, b_lin, gamma, beta):
    xc = x.reshape(_B, _C, _N)
    out = pl.pallas_call(
        _fused_kernel,
        grid=(_B + _B // 2,),
        in_specs=[
            pl.BlockSpec((1, _C, _N),
                         lambda s: (jnp.minimum(s, _B - 1), 0, 0)),
            pl.BlockSpec((_C, _C), lambda s: (0, 0)),
            pl.BlockSpec((1, _C), lambda s: (0, 0)),
            pl.BlockSpec((1, _C), lambda s: (0, 0)),
            pl.BlockSpec((1, _C), lambda s: (0, 0)),
            pl.BlockSpec((2, _C, _N),
                         lambda s: (jnp.maximum(s - _B, 0), 0, 0)),
        ],
        out_specs=pl.BlockSpec((2, _C, _N),
                               lambda s: (jnp.maximum(s - _B, 0), 0, 0)),
        out_shape=jax.ShapeDtypeStruct((_B, _C, _N), jnp.float32),
        scratch_shapes=[
            pltpu.VMEM((_B, _C, _N), jnp.float32),
            pltpu.VMEM((_B, _C, 2), jnp.float32),
        ],
    )(xc, W_lin, b_lin.reshape(1, _C), gamma.reshape(1, _C),
      beta.reshape(1, _C), xc)
    return out.reshape(_B, _C, _H, _W)
